# baseline (speedup 1.0000x reference)
import jax
import jax.numpy as jnp
from jax.experimental import pallas as pl

_N = 10000
_NPAD = 10240
_F_IN = 128
_F_OUT = 32
_BLOCK = 2560


def _body(x_ref, wiT_ref, wcT_ref, woT_ref, bi_ref, bhi_ref, bii_ref,
          bc_ref, bhc_ref, bcc_ref, bo_ref, bho_ref, boo_ref,
          wco_ref, wlinT_ref, blin_ref, out_ref):
    w = jnp.concatenate(
        [0.5 * wiT_ref[:], wcT_ref[:], 0.5 * woT_ref[:]], axis=0)
    brow = jnp.concatenate(
        [0.5 * (bi_ref[:] + bhi_ref[:] + bii_ref[:]),
         bc_ref[:] + bhc_ref[:] + bcc_ref[:],
         0.5 * (bo_ref[:] + bho_ref[:] + boo_ref[:])], axis=1)
    bcol = brow.reshape(3 * _F_OUT, 1)
    pT = jax.lax.dot_general(
        w, x_ref[:], (((1,), (1,)), ((), ())),
        preferred_element_type=jnp.float32) + bcol
    t1 = jnp.tanh(pT[:2 * _F_OUT, :])
    I = 0.5 + 0.5 * t1[:_F_OUT, :]
    C = I * t1[_F_OUT:, :]
    wco_col = (0.5 * wco_ref[:]).reshape(_F_OUT, 1)
    z2 = jnp.concatenate([pT[2 * _F_OUT:, :] + wco_col * C, C], axis=0)
    t2 = jnp.tanh(z2)
    O = 0.5 + 0.5 * t2[:_F_OUT, :]
    h = jnp.maximum(O * t2[_F_OUT:, :], 0.0)
    out_ref[:] = jax.lax.dot_general(
        wlinT_ref[:], h, (((1,), (0,)), ((), ())),
        preferred_element_type=jnp.float32) + blin_ref[:]


def kernel(x, edge_index, edge_weight,
           W_x_i, b_x_i, W_h_i, b_h_i, b_i,
           W_x_f, b_x_f, W_h_f, b_h_f, b_f,
           W_x_c, b_x_c, W_h_c, b_h_c, b_c,
           W_x_o, b_x_o, W_h_o, b_h_o, b_o,
           w_c_i, w_c_f, w_c_o, W_lin, b_lin):
    del edge_index, edge_weight, W_h_i, W_h_f, W_h_c, W_h_o
    del W_x_f, b_x_f, b_h_f, b_f, w_c_i, w_c_f
    row = lambda v: v.reshape(1, _F_OUT)

    rep = lambda shape: pl.BlockSpec(shape, lambda i: (0, 0))
    wspec = rep((_F_OUT, _F_IN))
    bspec = rep((1, _F_OUT))
    out2 = pl.pallas_call(
        _body,
        grid=(_NPAD // _BLOCK,),
        in_specs=[
            pl.BlockSpec((_BLOCK, _F_IN), lambda i: (i, 0)),
            wspec, wspec, wspec,
            bspec, bspec, bspec, bspec, bspec, bspec,
            bspec, bspec, bspec,
            bspec, bspec, rep((1, 1)),
        ],
        out_specs=pl.BlockSpec((1, _BLOCK), lambda i: (0, i)),
        out_shape=jax.ShapeDtypeStruct((1, _N), jnp.float32),
    )(x, W_x_i.T, W_x_c.T, W_x_o.T,
      row(b_x_i), row(b_h_i), row(b_i),
      row(b_x_c), row(b_h_c), row(b_c),
      row(b_x_o), row(b_h_o), row(b_o),
      row(w_c_o), W_lin.T, b_lin.reshape(1, 1))
    return out2.reshape(_N, 1)


# speedup vs baseline: 1.1609x; 1.1609x over previous
import jax
import jax.numpy as jnp
from jax.experimental import pallas as pl

_N = 10000
_NPAD = 10240
_F_IN = 128
_F_OUT = 32
_BLOCK = 10240


def _body(x_ref, wiT_ref, wcT_ref, woT_ref, bi_ref, bhi_ref, bii_ref,
          bc_ref, bhc_ref, bcc_ref, bo_ref, bho_ref, boo_ref,
          wco_ref, wlinT_ref, blin_ref, out_ref):
    w = jnp.concatenate(
        [0.5 * wiT_ref[:], wcT_ref[:], 0.5 * woT_ref[:]], axis=0)
    brow = jnp.concatenate(
        [0.5 * (bi_ref[:] + bhi_ref[:] + bii_ref[:]),
         bc_ref[:] + bhc_ref[:] + bcc_ref[:],
         0.5 * (bo_ref[:] + bho_ref[:] + boo_ref[:])], axis=1)
    bcol = brow.reshape(3 * _F_OUT, 1)
    pT = jax.lax.dot_general(
        w, x_ref[:], (((1,), (1,)), ((), ())),
        preferred_element_type=jnp.float32) + bcol
    t1 = jnp.tanh(pT[:2 * _F_OUT, :])
    I = 0.5 + 0.5 * t1[:_F_OUT, :]
    C = I * t1[_F_OUT:, :]
    wco_col = (0.5 * wco_ref[:]).reshape(_F_OUT, 1)
    z2 = jnp.concatenate([pT[2 * _F_OUT:, :] + wco_col * C, C], axis=0)
    t2 = jnp.tanh(z2)
    O = 0.5 + 0.5 * t2[:_F_OUT, :]
    h = jnp.maximum(O * t2[_F_OUT:, :], 0.0)
    out_ref[:] = jax.lax.dot_general(
        wlinT_ref[:], h, (((1,), (0,)), ((), ())),
        preferred_element_type=jnp.float32) + blin_ref[:]


def kernel(x, edge_index, edge_weight,
           W_x_i, b_x_i, W_h_i, b_h_i, b_i,
           W_x_f, b_x_f, W_h_f, b_h_f, b_f,
           W_x_c, b_x_c, W_h_c, b_h_c, b_c,
           W_x_o, b_x_o, W_h_o, b_h_o, b_o,
           w_c_i, w_c_f, w_c_o, W_lin, b_lin):
    del edge_index, edge_weight, W_h_i, W_h_f, W_h_c, W_h_o
    del W_x_f, b_x_f, b_h_f, b_f, w_c_i, w_c_f
    row = lambda v: v.reshape(1, _F_OUT)

    rep = lambda shape: pl.BlockSpec(shape, lambda i: (0, 0))
    wspec = rep((_F_OUT, _F_IN))
    bspec = rep((1, _F_OUT))
    out2 = pl.pallas_call(
        _body,
        grid=(_NPAD // _BLOCK,),
        in_specs=[
            pl.BlockSpec((_BLOCK, _F_IN), lambda i: (i, 0)),
            wspec, wspec, wspec,
            bspec, bspec, bspec, bspec, bspec, bspec,
            bspec, bspec, bspec,
            bspec, bspec, rep((1, 1)),
        ],
        out_specs=pl.BlockSpec((1, _BLOCK), lambda i: (0, i)),
        out_shape=jax.ShapeDtypeStruct((1, _N), jnp.float32),
    )(x, W_x_i.T, W_x_c.T, W_x_o.T,
      row(b_x_i), row(b_h_i), row(b_i),
      row(b_x_c), row(b_h_c), row(b_c),
      row(b_x_o), row(b_h_o), row(b_o),
      row(w_c_o), W_lin.T, b_lin.reshape(1, 1))
    return out2.reshape(_N, 1)


# speedup vs baseline: 1.2321x; 1.0613x over previous
import jax
import jax.numpy as jnp
from jax.experimental import pallas as pl

_N = 10000
_NPAD = 10240
_F_IN = 128
_F_OUT = 32
_BLOCK = 5120


def _body(x_ref, wiT_ref, wcT_ref, woT_ref, bi_ref, bhi_ref, bii_ref,
          bc_ref, bhc_ref, bcc_ref, bo_ref, bho_ref, boo_ref,
          wco_ref, wlinT_ref, blin_ref, out_ref):
    w = jnp.concatenate(
        [0.5 * wiT_ref[:], wcT_ref[:], 0.5 * woT_ref[:]], axis=0)
    brow = jnp.concatenate(
        [0.5 * (bi_ref[:] + bhi_ref[:] + bii_ref[:]),
         bc_ref[:] + bhc_ref[:] + bcc_ref[:],
         0.5 * (bo_ref[:] + bho_ref[:] + boo_ref[:])], axis=1)
    bcol = brow.reshape(3 * _F_OUT, 1)
    pT = jax.lax.dot_general(
        w, x_ref[:], (((1,), (1,)), ((), ())),
        preferred_element_type=jnp.float32) + bcol
    t1 = jnp.tanh(pT[:2 * _F_OUT, :])
    I = 0.5 + 0.5 * t1[:_F_OUT, :]
    C = I * t1[_F_OUT:, :]
    wco_col = (0.5 * wco_ref[:]).reshape(_F_OUT, 1)
    z2 = jnp.concatenate([pT[2 * _F_OUT:, :] + wco_col * C, C], axis=0)
    t2 = jnp.tanh(z2)
    O = 0.5 + 0.5 * t2[:_F_OUT, :]
    h = jnp.maximum(O * t2[_F_OUT:, :], 0.0)
    out_ref[:] = jax.lax.dot_general(
        wlinT_ref[:], h, (((1,), (0,)), ((), ())),
        preferred_element_type=jnp.float32) + blin_ref[:]


def kernel(x, edge_index, edge_weight,
           W_x_i, b_x_i, W_h_i, b_h_i, b_i,
           W_x_f, b_x_f, W_h_f, b_h_f, b_f,
           W_x_c, b_x_c, W_h_c, b_h_c, b_c,
           W_x_o, b_x_o, W_h_o, b_h_o, b_o,
           w_c_i, w_c_f, w_c_o, W_lin, b_lin):
    del edge_index, edge_weight, W_h_i, W_h_f, W_h_c, W_h_o
    del W_x_f, b_x_f, b_h_f, b_f, w_c_i, w_c_f
    row = lambda v: v.reshape(1, _F_OUT)

    rep = lambda shape: pl.BlockSpec(shape, lambda i: (0, 0))
    wspec = rep((_F_OUT, _F_IN))
    bspec = rep((1, _F_OUT))
    out2 = pl.pallas_call(
        _body,
        grid=(_NPAD // _BLOCK,),
        in_specs=[
            pl.BlockSpec((_BLOCK, _F_IN), lambda i: (i, 0)),
            wspec, wspec, wspec,
            bspec, bspec, bspec, bspec, bspec, bspec,
            bspec, bspec, bspec,
            bspec, bspec, rep((1, 1)),
        ],
        out_specs=pl.BlockSpec((1, _BLOCK), lambda i: (0, i)),
        out_shape=jax.ShapeDtypeStruct((1, _N), jnp.float32),
    )(x, W_x_i.T, W_x_c.T, W_x_o.T,
      row(b_x_i), row(b_h_i), row(b_i),
      row(b_x_c), row(b_h_c), row(b_c),
      row(b_x_o), row(b_h_o), row(b_o),
      row(w_c_o), W_lin.T, b_lin.reshape(1, 1))
    return out2.reshape(_N, 1)
